# baseline (speedup 1.0000x reference)
import jax
import jax.numpy as jnp
from jax.experimental import pallas as pl

_S = 576
_T = 32768
_TOPK = 1
_TGT = 577
_NPAD = 640
_D = 1280
_SENTINEL = 2_000_000
_BLK = 192


def _stage1_body(x_ref, vals_ref, idx_ref):
    x = x_ref[...]
    m = jnp.max(x, axis=1, keepdims=True)
    col = jax.lax.broadcasted_iota(jnp.int32, x.shape, 1)
    am = jnp.min(jnp.where(x == m, col, _T), axis=1, keepdims=True)
    vals_ref[...] = m
    idx_ref[...] = am


def _stage2_body(idx_ref, vals_ref, topk_ref, tgt_ref, out_ref, vrow_ref):
    sv_col = jax.lax.broadcasted_iota(jnp.int32, (_S, 1), 0)
    sv_row = jax.lax.broadcasted_iota(jnp.int32, (1, _S), 1)
    eye_s = (sv_col == sv_row).astype(jnp.float32)
    vrow_ref[...] = jax.lax.dot_general(
        vals_ref[...], eye_s, (((0,), (0,)), ((), ())),
        precision=jax.lax.Precision.HIGHEST,
        preferred_element_type=jnp.float32)
    shift = 1 + (topk_ref[0, 0] - _TOPK)
    tail = jax.lax.broadcasted_iota(jnp.int32, (_NPAD - _S, 1), 0)
    tail = jnp.where(tail == 0, 0, _SENTINEL)
    q_col = jnp.concatenate([idx_ref[...] + shift, tail], axis=0)
    s_col = jax.lax.broadcasted_iota(jnp.int32, (_NPAD, 1), 0)
    s_row = jax.lax.broadcasted_iota(jnp.int32, (1, _NPAD), 1)
    eye = (s_col == s_row).astype(jnp.float32)
    q_row = jax.lax.dot_general(
        q_col.astype(jnp.float32), eye, (((0,), (0,)), ((), ())),
        precision=jax.lax.Precision.HIGHEST,
        preferred_element_type=jnp.float32).astype(jnp.int32)
    valid_col = s_col < _TGT
    valid_row = s_row < _TGT

    eq = (q_col == q_row).astype(jnp.int32)
    lt_ct = (s_col < s_row).astype(jnp.int32)
    lt_tc = (s_row < s_col).astype(jnp.int32)
    u_row = ((jnp.sum(eq * lt_ct, axis=0, keepdims=True) == 0)
             & valid_row).astype(jnp.int32)
    u_col = ((jnp.sum(eq * lt_tc, axis=1, keepdims=True) == 0)
             & valid_col).astype(jnp.int32)

    p_cnt = jnp.sum(u_row)
    k_pad = _TGT - p_cnt

    i_col = jax.lax.broadcasted_iota(jnp.int32, (_D, 1), 0)
    le = (q_row <= i_col).astype(jnp.int32)
    oc = jnp.sum(le * u_row, axis=1, keepdims=True)
    cs = oc + jnp.minimum(k_pad, (i_col + 1) - oc)
    cs_d = jnp.sum(u_row * (q_row < _D)) + k_pad

    j_row = s_row
    out_small = jnp.sum((cs <= j_row).astype(jnp.int32), axis=0,
                        keepdims=True)

    b_row = u_row * (q_row >= _D).astype(jnp.int32)
    b_col = u_col * (q_col >= _D).astype(jnp.int32)
    r_col = cs_d + jnp.sum((q_row < q_col).astype(jnp.int32) * b_row,
                           axis=1, keepdims=True)
    hit = (r_col == j_row).astype(jnp.int32) * b_col
    out_big = jnp.sum(hit * q_col, axis=0, keepdims=True)

    picked = (jnp.where(j_row < cs_d, out_small, out_big)
              + (tgt_ref[0, 0] - _TGT))
    out_ref[...] = picked.reshape(_NPAD)[:_TGT]


def _run(attn2d, target_num, top_k):
    vals, idx = pl.pallas_call(
        _stage1_body,
        grid=(_S // _BLK,),
        in_specs=[pl.BlockSpec((_BLK, _T), lambda i: (i, 0))],
        out_specs=[pl.BlockSpec((_BLK, 1), lambda i: (i, 0)),
                   pl.BlockSpec((_BLK, 1), lambda i: (i, 0))],
        out_shape=[jax.ShapeDtypeStruct((_S, 1), jnp.float32),
                   jax.ShapeDtypeStruct((_S, 1), jnp.int32)],
    )(attn2d)

    topk_arr = jnp.asarray(top_k, jnp.int32).reshape(1, 1)
    tgt_arr = jnp.asarray(target_num, jnp.int32).reshape(1, 1)
    picked, vrow = pl.pallas_call(
        _stage2_body,
        out_shape=[jax.ShapeDtypeStruct((_TGT,), jnp.int32),
                   jax.ShapeDtypeStruct((1, _S), jnp.float32)],
    )(idx, vals, topk_arr, tgt_arr)
    return vrow.reshape(1, _S, 1), picked


def kernel(attn_qk, target_num, top_k):
    if attn_qk.ndim == 2:
        attn_qk = attn_qk[None]
    return _run(attn_qk.reshape(_S, _T), target_num, top_k)


# speedup vs baseline: 1.1479x; 1.1479x over previous
import jax
import jax.numpy as jnp
from jax.experimental import pallas as pl
from jax.experimental.pallas import tpu as pltpu

_S = 576
_T = 32768
_TOPK = 1
_TGT = 577
_NPAD = 640
_D = 1280
_SENTINEL = 2_000_000
_BLK = 96


def _fused_body(x_ref, topk_ref, tgt_ref, vrow_ref, out_ref,
                vals_s, idx_s):
    i = pl.program_id(0)
    x = x_ref[...]
    m = jnp.max(x, axis=1, keepdims=True)
    col = jax.lax.broadcasted_iota(jnp.int32, x.shape, 1)
    am = jnp.min(jnp.where(x == m, col, _T), axis=1, keepdims=True)
    vals_s[pl.ds(i * _BLK, _BLK), :] = m
    idx_s[pl.ds(i * _BLK, _BLK), :] = am

    @pl.when(i == _S // _BLK - 1)
    def _stage2():
        _stage2_math(idx_s, vals_s, topk_ref, tgt_ref, out_ref, vrow_ref)


def _stage2_math(idx_ref, vals_ref, topk_ref, tgt_ref, out_ref, vrow_ref):
    sv_col = jax.lax.broadcasted_iota(jnp.int32, (_S, 1), 0)
    sv_row = jax.lax.broadcasted_iota(jnp.int32, (1, _S), 1)
    eye_s = (sv_col == sv_row).astype(jnp.float32)
    vrow_ref[...] = jax.lax.dot_general(
        vals_ref[...], eye_s, (((0,), (0,)), ((), ())),
        precision=jax.lax.Precision.HIGHEST,
        preferred_element_type=jnp.float32)
    shift = 1 + (topk_ref[0, 0] - _TOPK)
    tail = jax.lax.broadcasted_iota(jnp.int32, (_NPAD - _S, 1), 0)
    tail = jnp.where(tail == 0, 0, _SENTINEL)
    q_col = jnp.concatenate([idx_ref[...] + shift, tail], axis=0)
    s_col = jax.lax.broadcasted_iota(jnp.int32, (_NPAD, 1), 0)
    s_row = jax.lax.broadcasted_iota(jnp.int32, (1, _NPAD), 1)
    eye = (s_col == s_row).astype(jnp.float32)
    q_row = jax.lax.dot_general(
        q_col.astype(jnp.float32), eye, (((0,), (0,)), ((), ())),
        precision=jax.lax.Precision.HIGHEST,
        preferred_element_type=jnp.float32).astype(jnp.int32)
    valid_col = s_col < _TGT
    valid_row = s_row < _TGT

    eq = (q_col == q_row).astype(jnp.int32)
    lt_ct = (s_col < s_row).astype(jnp.int32)
    lt_tc = (s_row < s_col).astype(jnp.int32)
    u_row = ((jnp.sum(eq * lt_ct, axis=0, keepdims=True) == 0)
             & valid_row).astype(jnp.int32)
    u_col = ((jnp.sum(eq * lt_tc, axis=1, keepdims=True) == 0)
             & valid_col).astype(jnp.int32)

    p_cnt = jnp.sum(u_row)
    k_pad = _TGT - p_cnt

    i_col = jax.lax.broadcasted_iota(jnp.int32, (_D, 1), 0)
    le = (q_row <= i_col).astype(jnp.int32)
    oc = jnp.sum(le * u_row, axis=1, keepdims=True)
    cs = oc + jnp.minimum(k_pad, (i_col + 1) - oc)
    cs_d = jnp.sum(u_row * (q_row < _D)) + k_pad

    j_row = s_row
    out_small = jnp.sum((cs <= j_row).astype(jnp.int32), axis=0,
                        keepdims=True)

    b_row = u_row * (q_row >= _D).astype(jnp.int32)
    b_col = u_col * (q_col >= _D).astype(jnp.int32)
    r_col = cs_d + jnp.sum((q_row < q_col).astype(jnp.int32) * b_row,
                           axis=1, keepdims=True)
    hit = (r_col == j_row).astype(jnp.int32) * b_col
    out_big = jnp.sum(hit * q_col, axis=0, keepdims=True)

    picked = (jnp.where(j_row < cs_d, out_small, out_big)
              + (tgt_ref[0, 0] - _TGT))
    out_ref[...] = picked.reshape(_NPAD)[:_TGT]


def _run(attn2d, target_num, top_k):
    topk_arr = jnp.asarray(top_k, jnp.int32).reshape(1, 1)
    tgt_arr = jnp.asarray(target_num, jnp.int32).reshape(1, 1)
    vrow, picked = pl.pallas_call(
        _fused_body,
        grid=(_S // _BLK,),
        in_specs=[pl.BlockSpec((_BLK, _T), lambda i: (i, 0)),
                  pl.BlockSpec((1, 1), lambda i: (0, 0)),
                  pl.BlockSpec((1, 1), lambda i: (0, 0))],
        out_specs=[pl.BlockSpec((1, _S), lambda i: (0, 0)),
                   pl.BlockSpec((_TGT,), lambda i: (0,))],
        out_shape=[jax.ShapeDtypeStruct((1, _S), jnp.float32),
                   jax.ShapeDtypeStruct((_TGT,), jnp.int32)],
        scratch_shapes=[pltpu.VMEM((_S, 1), jnp.float32),
                        pltpu.VMEM((_S, 1), jnp.int32)],
    )(attn2d, topk_arr, tgt_arr)
    return vrow.reshape(1, _S, 1), picked


def kernel(attn_qk, target_num, top_k):
    if attn_qk.ndim == 2:
        attn_qk = attn_qk[None]
    return _run(attn_qk.reshape(_S, _T), target_num, top_k)


# speedup vs baseline: 1.1684x; 1.0179x over previous
import jax
import jax.numpy as jnp
from jax.experimental import pallas as pl
from jax.experimental.pallas import tpu as pltpu

_S = 576
_T = 32768
_TOPK = 1
_TGT = 577
_NPAD = 640
_D = 640
_SENTINEL = 2_000_000
_BLK = 96


def _fused_body(x_ref, topk_ref, tgt_ref, vrow_ref, out_ref,
                vals_s, idx_s):
    i = pl.program_id(0)
    x = x_ref[...]
    m = jnp.max(x, axis=1, keepdims=True)
    col = jax.lax.broadcasted_iota(jnp.int32, x.shape, 1)
    am = jnp.min(jnp.where(x == m, col, _T), axis=1, keepdims=True)
    vals_s[pl.ds(i * _BLK, _BLK), :] = m
    idx_s[pl.ds(i * _BLK, _BLK), :] = am

    @pl.when(i == _S // _BLK - 1)
    def _stage2():
        _stage2_math(idx_s, vals_s, topk_ref, tgt_ref, out_ref, vrow_ref)


def _stage2_math(idx_ref, vals_ref, topk_ref, tgt_ref, out_ref, vrow_ref):
    sv_col = jax.lax.broadcasted_iota(jnp.int32, (_S, 1), 0)
    sv_row = jax.lax.broadcasted_iota(jnp.int32, (1, _S), 1)
    eye_s = (sv_col == sv_row).astype(jnp.float32)
    vrow_ref[...] = jax.lax.dot_general(
        vals_ref[...], eye_s, (((0,), (0,)), ((), ())),
        precision=jax.lax.Precision.HIGHEST,
        preferred_element_type=jnp.float32)
    shift = 1 + (topk_ref[0, 0] - _TOPK)
    tail = jax.lax.broadcasted_iota(jnp.int32, (_NPAD - _S, 1), 0)
    tail = jnp.where(tail == 0, 0, _SENTINEL)
    q_col = jnp.concatenate([idx_ref[...] + shift, tail], axis=0)
    s_col = jax.lax.broadcasted_iota(jnp.int32, (_NPAD, 1), 0)
    s_row = jax.lax.broadcasted_iota(jnp.int32, (1, _NPAD), 1)
    eye = (s_col == s_row).astype(jnp.float32)
    q_row = jax.lax.dot_general(
        q_col.astype(jnp.float32), eye, (((0,), (0,)), ((), ())),
        precision=jax.lax.Precision.HIGHEST,
        preferred_element_type=jnp.float32).astype(jnp.int32)
    valid_col = s_col < _TGT
    valid_row = s_row < _TGT

    eq = (q_col == q_row).astype(jnp.int32)
    lt_ct = (s_col < s_row).astype(jnp.int32)
    lt_tc = (s_row < s_col).astype(jnp.int32)
    u_row = ((jnp.sum(eq * lt_ct, axis=0, keepdims=True) == 0)
             & valid_row).astype(jnp.int32)
    u_col = ((jnp.sum(eq * lt_tc, axis=1, keepdims=True) == 0)
             & valid_col).astype(jnp.int32)

    p_cnt = jnp.sum(u_row)
    k_pad = _TGT - p_cnt

    i_col = jax.lax.broadcasted_iota(jnp.int32, (_D, 1), 0)
    le = (q_row <= i_col).astype(jnp.int32)
    oc = jnp.sum(le * u_row, axis=1, keepdims=True)
    cs = oc + jnp.minimum(k_pad, (i_col + 1) - oc)
    cs_d = jnp.sum(u_row * (q_row < _D)) + k_pad

    j_row = s_row
    out_small = jnp.sum((cs <= j_row).astype(jnp.int32), axis=0,
                        keepdims=True)

    b_row = u_row * (q_row >= _D).astype(jnp.int32)
    b_col = u_col * (q_col >= _D).astype(jnp.int32)
    r_col = cs_d + jnp.sum((q_row < q_col).astype(jnp.int32) * b_row,
                           axis=1, keepdims=True)
    hit = (r_col == j_row).astype(jnp.int32) * b_col
    out_big = jnp.sum(hit * q_col, axis=0, keepdims=True)

    picked = (jnp.where(j_row < cs_d, out_small, out_big)
              + (tgt_ref[0, 0] - _TGT))
    out_ref[...] = picked.reshape(_NPAD)[:_TGT]


def _run(attn2d, target_num, top_k):
    topk_arr = jnp.asarray(top_k, jnp.int32).reshape(1, 1)
    tgt_arr = jnp.asarray(target_num, jnp.int32).reshape(1, 1)
    vrow, picked = pl.pallas_call(
        _fused_body,
        grid=(_S // _BLK,),
        in_specs=[pl.BlockSpec((_BLK, _T), lambda i: (i, 0)),
                  pl.BlockSpec((1, 1), lambda i: (0, 0)),
                  pl.BlockSpec((1, 1), lambda i: (0, 0))],
        out_specs=[pl.BlockSpec((1, _S), lambda i: (0, 0)),
                   pl.BlockSpec((_TGT,), lambda i: (0,))],
        out_shape=[jax.ShapeDtypeStruct((1, _S), jnp.float32),
                   jax.ShapeDtypeStruct((_TGT,), jnp.int32)],
        scratch_shapes=[pltpu.VMEM((_S, 1), jnp.float32),
                        pltpu.VMEM((_S, 1), jnp.int32)],
    )(attn2d, topk_arr, tgt_arr)
    return vrow.reshape(1, _S, 1), picked


def kernel(attn_qk, target_num, top_k):
    if attn_qk.ndim == 2:
        attn_qk = attn_qk[None]
    return _run(attn_qk.reshape(_S, _T), target_num, top_k)
